# baseline (speedup 1.0000x reference)
import functools

import jax
import jax.numpy as jnp
from jax.experimental import pallas as pl

N, F_IN, H, F_OUT = 10000, 128, 24, 16
BM = 512


def _proj_kernel(x_ref, w_ref, o_ref):
    o_ref[...] = jnp.dot(x_ref[...], w_ref[...],
                         preferred_element_type=jnp.float32)


def _layer1_kernel(adj_ref, s1_ref, b1_ref, w2_ref, o_ref):
    t = jnp.dot(adj_ref[...], s1_ref[...], preferred_element_type=jnp.float32)
    h = jnp.maximum(t + b1_ref[...], 0.0)
    o_ref[...] = jnp.dot(h, w2_ref[...], preferred_element_type=jnp.float32)


def _layer2_kernel(adj_ref, s2_ref, b2_ref, o_ref):
    t = jnp.dot(adj_ref[...], s2_ref[...], preferred_element_type=jnp.float32)
    o_ref[...] = t + b2_ref[...]


@functools.partial(jax.jit, static_argnames=())
def kernel(x, adj, W1, b1, W2, b2):
    n = adj.shape[0]
    grid = (pl.cdiv(n, BM),)

    s1 = pl.pallas_call(
        _proj_kernel,
        out_shape=jax.ShapeDtypeStruct((n, H), jnp.float32),
        in_specs=[
            pl.BlockSpec((BM, F_IN), lambda i: (i, 0)),
            pl.BlockSpec((F_IN, H), lambda i: (0, 0)),
        ],
        out_specs=pl.BlockSpec((BM, H), lambda i: (i, 0)),
        grid=grid,
    )(x, W1)

    b1_2d = b1.reshape(1, H)
    s2 = pl.pallas_call(
        _layer1_kernel,
        out_shape=jax.ShapeDtypeStruct((n, F_OUT), jnp.float32),
        in_specs=[
            pl.BlockSpec((BM, n), lambda i: (i, 0)),
            pl.BlockSpec((n, H), lambda i: (0, 0)),
            pl.BlockSpec((1, H), lambda i: (0, 0)),
            pl.BlockSpec((H, F_OUT), lambda i: (0, 0)),
        ],
        out_specs=pl.BlockSpec((BM, F_OUT), lambda i: (i, 0)),
        grid=grid,
    )(adj, s1, b1_2d, W2)

    b2_2d = b2.reshape(1, F_OUT)
    out = pl.pallas_call(
        _layer2_kernel,
        out_shape=jax.ShapeDtypeStruct((n, F_OUT), jnp.float32),
        in_specs=[
            pl.BlockSpec((BM, n), lambda i: (i, 0)),
            pl.BlockSpec((n, F_OUT), lambda i: (0, 0)),
            pl.BlockSpec((1, F_OUT), lambda i: (0, 0)),
        ],
        out_specs=pl.BlockSpec((BM, F_OUT), lambda i: (i, 0)),
        grid=grid,
    )(adj, s2, b2_2d)

    return out
